# baseline (speedup 1.0000x reference)
import jax
import jax.numpy as jnp
from jax import lax
from jax.experimental import pallas as pl
from jax.experimental.pallas import tpu as pltpu
from jax.experimental.pallas import tpu_sc as plsc

N_TOK = 2048
D_EMB = 4096
D_F = 2048
D_S = 1024
NQ = 8
K = 1024
MT = 256
NMT = N_TOK // MT

H = N_TOK // 2
NHT = H // MT

SC_NC = 2
SC_NS = 16
ROWS_PER_W = H // (SC_NC * SC_NS)


def _encoder_body(x_ref, w1_ref, b1_ref, w2_ref, b2_ref, z_ref, m_ref):
    h = jnp.dot(x_ref[...].astype(jnp.bfloat16), w1_ref[...],
                preferred_element_type=jnp.float32)
    h = h + b1_ref[...]
    z = jnp.dot(h.astype(jnp.bfloat16), w2_ref[...],
                preferred_element_type=jnp.float32)
    z = z + b2_ref[...]
    z_ref[...] = z
    m_ref[...] = -z


def _encoder(x, w1_16, b1, w2_16, b2):
    return pl.pallas_call(
        _encoder_body,
        grid=(NMT,),
        in_specs=[
            pl.BlockSpec((MT, D_EMB), lambda i: (i, 0)),
            pl.BlockSpec((D_EMB, D_F), lambda i: (0, 0)),
            pl.BlockSpec((1, D_F), lambda i: (0, 0)),
            pl.BlockSpec((D_F, D_S), lambda i: (0, 0)),
            pl.BlockSpec((1, D_S), lambda i: (0, 0)),
        ],
        out_specs=[
            pl.BlockSpec((MT, D_S), lambda i: (i, 0)),
            pl.BlockSpec((MT, D_S), lambda i: (i, 0)),
        ],
        out_shape=[
            jax.ShapeDtypeStruct((N_TOK, D_S), jnp.float32),
            jax.ShapeDtypeStruct((N_TOK, D_S), jnp.float32),
        ],
    )(x, w1_16, b1, w2_16, b2)


def _prep_body(cb_ref, cbt_ref):
    cbt_ref[0] = cb_ref[0].T.astype(jnp.bfloat16)


def _prep(codebooks):
    return pl.pallas_call(
        _prep_body,
        grid=(NQ,),
        in_specs=[pl.BlockSpec((1, K, D_S), lambda q: (q, 0, 0))],
        out_specs=pl.BlockSpec((1, D_S, K), lambda q: (q, 0, 0)),
        out_shape=jax.ShapeDtypeStruct((NQ, D_S, K), jnp.bfloat16),
    )(codebooks)


def _fdk_body(q, m_ref, rsq_ref, cbt_ref, cbn_ref, idx_ref, ioff_ref,
              loss_ref):
    s = jnp.dot(m_ref[...].astype(jnp.bfloat16), cbt_ref[0],
                preferred_element_type=jnp.float32)
    dist = (rsq_ref[0] + 2.0 * s) + cbn_ref[0]
    mn = jnp.min(dist, axis=1, keepdims=True)
    ii = lax.broadcasted_iota(jnp.int32, (MT, K), 1)
    idx = jnp.min(jnp.where(dist == mn, ii, K), axis=1)
    idx_ref[0, 0, :] = idx
    ioff_ref[0, 0, :] = idx + q * K

    @pl.when(pl.program_id(0) == 0)
    def _():
        loss_ref[...] = jnp.zeros_like(loss_ref)

    loss_ref[...] += (jnp.sum(rsq_ref[0]) * (1.0 / (N_TOK * D_S))
                      ).reshape(1, 1)


def _fdku_body(q, m_ref, q_ref, rsq_ref, cbt_ref, cbn_ref, mn_ref,
               idx_ref, ioff_ref, loss_ref):
    mn = m_ref[...] + q_ref[...]
    mn_ref[...] = mn
    s = jnp.dot(mn.astype(jnp.bfloat16), cbt_ref[0],
                preferred_element_type=jnp.float32)
    dist = (rsq_ref[0] + 2.0 * s) + cbn_ref[0]
    mnv = jnp.min(dist, axis=1, keepdims=True)
    ii = lax.broadcasted_iota(jnp.int32, (MT, K), 1)
    idx = jnp.min(jnp.where(dist == mnv, ii, K), axis=1)
    idx_ref[0, 0, :] = idx
    ioff_ref[0, 0, :] = idx + q * K

    @pl.when(pl.program_id(0) == 0)
    def _():
        loss_ref[...] = jnp.zeros_like(loss_ref)

    loss_ref[...] += (jnp.sum(rsq_ref[0]) * (1.0 / (N_TOK * D_S))
                      ).reshape(1, 1)


def _fdk(m, rsq, cbt, cbn, q, off):
    return pl.pallas_call(
        lambda *a: _fdk_body(q, *a),
        grid=(NHT,),
        in_specs=[
            pl.BlockSpec((MT, D_S), lambda i, o=off: (i + o * NHT, 0)),
            pl.BlockSpec((1, MT, 1), lambda i, o=off: (i + o * NHT, 0, 0)),
            pl.BlockSpec((1, D_S, K), lambda i, q=q: (q, 0, 0)),
            pl.BlockSpec((1, 1, K), lambda i, q=q: (q, 0, 0)),
        ],
        out_specs=[
            pl.BlockSpec((1, 1, MT), lambda i: (i, 0, 0)),
            pl.BlockSpec((1, 1, MT), lambda i: (i, 0, 0)),
            pl.BlockSpec((1, 1), lambda i: (0, 0)),
        ],
        out_shape=[
            jax.ShapeDtypeStruct((NHT, 1, MT), jnp.int32),
            jax.ShapeDtypeStruct((NHT, 1, MT), jnp.int32),
            jax.ShapeDtypeStruct((1, 1), jnp.float32),
        ],
    )(m, rsq, cbt, cbn)


def _fdku(m_prev, quant, rsq, cbt, cbn, q, moff):
    return pl.pallas_call(
        lambda *a: _fdku_body(q, *a),
        grid=(NHT,),
        in_specs=[
            pl.BlockSpec((MT, D_S), lambda i, o=moff: (i + o, 0)),
            pl.BlockSpec((MT, D_S), lambda i: (i, 0)),
            pl.BlockSpec((1, MT, 1), lambda i: (i, 0, 0)),
            pl.BlockSpec((1, D_S, K), lambda i, q=q: (q, 0, 0)),
            pl.BlockSpec((1, 1, K), lambda i, q=q: (q, 0, 0)),
        ],
        out_specs=[
            pl.BlockSpec((MT, D_S), lambda i: (i, 0)),
            pl.BlockSpec((1, 1, MT), lambda i: (i, 0, 0)),
            pl.BlockSpec((1, 1, MT), lambda i: (i, 0, 0)),
            pl.BlockSpec((1, 1), lambda i: (0, 0)),
        ],
        out_shape=[
            jax.ShapeDtypeStruct((H, D_S), jnp.float32),
            jax.ShapeDtypeStruct((NHT, 1, MT), jnp.int32),
            jax.ShapeDtypeStruct((NHT, 1, MT), jnp.int32),
            jax.ShapeDtypeStruct((1, 1), jnp.float32),
        ],
    )(m_prev, quant, rsq, cbt, cbn)


def _scupd_body(cb_hbm, ioff_hbm, out_hbm, idx_v, rows_v, sem):
    wid = lax.axis_index("s") * SC_NC + lax.axis_index("c")
    base = wid * ROWS_PER_W
    pltpu.sync_copy(ioff_hbm.at[pl.ds(base, ROWS_PER_W)], idx_v)
    pltpu.async_copy(cb_hbm.at[idx_v], rows_v, sem).wait()
    pltpu.sync_copy(rows_v, out_hbm.at[pl.ds(base, ROWS_PER_W)])


_SC_UPD_KERNEL = None


def _sc_gather(cb_flat, ioff):
    global _SC_UPD_KERNEL
    if _SC_UPD_KERNEL is None:
        _SC_UPD_KERNEL = pl.kernel(
            _scupd_body,
            out_type=jax.ShapeDtypeStruct((H, D_S), jnp.float32),
            mesh=plsc.VectorSubcoreMesh(core_axis_name="c",
                                        subcore_axis_name="s",
                                        num_cores=SC_NC,
                                        num_subcores=SC_NS),
            scratch_types=[
                pltpu.VMEM((ROWS_PER_W,), jnp.int32),
                pltpu.VMEM((ROWS_PER_W, D_S), jnp.float32),
                pltpu.SemaphoreType.DMA,
            ],
        )
    return _SC_UPD_KERNEL(cb_flat, ioff)


def _decoder_body(z_ref, m_ref, q_ref, w3_ref, b3_ref, w4_ref, b4_ref,
                  out_ref, loss_ref):
    m = m_ref[...] + q_ref[...]

    @pl.when(pl.program_id(0) == 0)
    def _():
        loss_ref[...] = jnp.zeros_like(loss_ref)

    loss_ref[...] += (jnp.sum(m * m) * (1.0 / (N_TOK * D_S))).reshape(1, 1)
    zq = (z_ref[...] + m).astype(jnp.bfloat16)
    h = jnp.dot(zq, w3_ref[...], preferred_element_type=jnp.float32)
    h = h + b3_ref[...]
    o = jnp.dot(h.astype(jnp.bfloat16), w4_ref[...],
                preferred_element_type=jnp.float32)
    out_ref[...] = o + b4_ref[...]


def _decoder(z_e, m_last, quant_last, w3_16, b3, w4_16, b4, off):
    return pl.pallas_call(
        _decoder_body,
        grid=(NHT,),
        in_specs=[
            pl.BlockSpec((MT, D_S), lambda i, o=off: (i + o * NHT, 0)),
            pl.BlockSpec((MT, D_S), lambda i: (i, 0)),
            pl.BlockSpec((MT, D_S), lambda i: (i, 0)),
            pl.BlockSpec((D_S, D_F), lambda i: (0, 0)),
            pl.BlockSpec((1, D_F), lambda i: (0, 0)),
            pl.BlockSpec((D_F, D_EMB), lambda i: (0, 0)),
            pl.BlockSpec((1, D_EMB), lambda i: (0, 0)),
        ],
        out_specs=[
            pl.BlockSpec((MT, D_EMB), lambda i: (i, 0)),
            pl.BlockSpec((1, 1), lambda i: (0, 0)),
        ],
        out_shape=[
            jax.ShapeDtypeStruct((H, D_EMB), jnp.float32),
            jax.ShapeDtypeStruct((1, 1), jnp.float32),
        ],
    )(z_e, m_last, quant_last, w3_16, b3, w4_16, b4)


def _rsq_sum(m_prev, quant):
    return jnp.sum((m_prev + quant).reshape(1, H, D_S) ** 2, axis=-1,
                   keepdims=True).reshape(NHT, MT, 1)


def _rsq(m):
    return jnp.sum(m.reshape(1, N_TOK, D_S) ** 2, axis=-1,
                   keepdims=True).reshape(NMT, MT, 1)


def kernel(x, W1, b1, W2, b2, codebooks, W3, b3, W4, b4):
    z_e, m = _encoder(x.reshape(N_TOK, D_EMB),
                      W1.astype(jnp.bfloat16), b1.reshape(1, D_F),
                      W2.astype(jnp.bfloat16), b2.reshape(1, D_S))

    cbt = _prep(codebooks)
    cbn = jnp.sum(codebooks ** 2, axis=-1).reshape(NQ, 1, K)
    cb_flat = codebooks.reshape(NQ * K, D_S)

    rsq0 = _rsq(m)
    idxs = [[], []]
    losses = []

    ioff0 = [None, None]
    for h in range(2):
        idx0, ioff_h, _ = _fdk(m, rsq0, cbt, cbn, 0, h)
        idxs[h].append(idx0.reshape(H))
        ioff0[h] = ioff_h
    quant = [_sc_gather(cb_flat, ioff0[0].reshape(H)),
             _sc_gather(cb_flat, ioff0[1].reshape(H))]

    m_h = [m, m]
    moff = [0 * NHT, 1 * NHT]
    for q in range(1, NQ):
        step_losses = []
        ioffs = [None, None]
        for h in range(2):
            m_for_rsq = (m_h[h] if m_h[h].shape[0] == H else
                         lax.slice_in_dim(m_h[h], moff[h] * MT,
                                          moff[h] * MT + H))
            rsq = _rsq_sum(m_for_rsq, quant[h])
            mh_new, idx_q, ioff_q, loss_q = _fdku(
                m_h[h], quant[h], rsq, cbt, cbn, q, moff[h])
            m_h[h] = mh_new
            moff[h] = 0
            idxs[h].append(idx_q.reshape(H))
            step_losses.append(loss_q)
            ioffs[h] = ioff_q
        losses.append(step_losses)
        quant = [_sc_gather(cb_flat, ioffs[0].reshape(H)),
                 _sc_gather(cb_flat, ioffs[1].reshape(H))]

    outs = []
    dec_losses = []
    for h in range(2):
        o_h, l_h = _decoder(z_e, m_h[h], quant[h],
                            W3.astype(jnp.bfloat16), b3.reshape(1, D_F),
                            W4.astype(jnp.bfloat16), b4.reshape(1, D_EMB),
                            h)
        outs.append(o_h)
        dec_losses.append(l_h)
    losses.append(dec_losses)

    out = jnp.concatenate(outs, axis=0).reshape(1, N_TOK, D_EMB)
    indices = jnp.stack(
        [jnp.concatenate([idxs[0][q], idxs[1][q]]).reshape(1, N_TOK)
         for q in range(NQ)], axis=-1)
    cmt_loss = jnp.concatenate(
        [(la + lb).reshape(1) for la, lb in losses])
    return (out, indices, cmt_loss)


# speedup vs baseline: 1.1061x; 1.1061x over previous
import jax
import jax.numpy as jnp
from jax import lax
from jax.experimental import pallas as pl
from jax.experimental.pallas import tpu as pltpu
from jax.experimental.pallas import tpu_sc as plsc

N_TOK = 2048
D_EMB = 4096
D_F = 2048
D_S = 1024
NQ = 8
K = 1024
MT = 256
NMT = N_TOK // MT
MTQ = 512
NMTQ = N_TOK // MTQ

SC_NC = 2
SC_NS = 16
ROWS_PER_W = N_TOK // (SC_NC * SC_NS)


def _encoder_body(x_ref, w1_ref, b1_ref, w2_ref, b2_ref, z_ref, m_ref):
    h = jnp.dot(x_ref[...].astype(jnp.bfloat16), w1_ref[...],
                preferred_element_type=jnp.float32)
    h = h + b1_ref[...]
    z = jnp.dot(h.astype(jnp.bfloat16), w2_ref[...],
                preferred_element_type=jnp.float32)
    z = z + b2_ref[...]
    z_ref[...] = z
    m_ref[...] = -z


def _encoder(x, w1_16, b1, w2_16, b2):
    return pl.pallas_call(
        _encoder_body,
        grid=(NMT,),
        in_specs=[
            pl.BlockSpec((MT, D_EMB), lambda i: (i, 0)),
            pl.BlockSpec((D_EMB, D_F), lambda i: (0, 0)),
            pl.BlockSpec((1, D_F), lambda i: (0, 0)),
            pl.BlockSpec((D_F, D_S), lambda i: (0, 0)),
            pl.BlockSpec((1, D_S), lambda i: (0, 0)),
        ],
        out_specs=[
            pl.BlockSpec((MT, D_S), lambda i: (i, 0)),
            pl.BlockSpec((MT, D_S), lambda i: (i, 0)),
        ],
        out_shape=[
            jax.ShapeDtypeStruct((N_TOK, D_S), jnp.float32),
            jax.ShapeDtypeStruct((N_TOK, D_S), jnp.float32),
        ],
    )(x, w1_16, b1, w2_16, b2)


def _prep_body(cb_ref, cbt_ref):
    cbt_ref[0] = cb_ref[0].T.astype(jnp.bfloat16)


def _prep(codebooks):
    return pl.pallas_call(
        _prep_body,
        grid=(NQ,),
        in_specs=[pl.BlockSpec((1, K, D_S), lambda q: (q, 0, 0))],
        out_specs=pl.BlockSpec((1, D_S, K), lambda q: (q, 0, 0)),
        out_shape=jax.ShapeDtypeStruct((NQ, D_S, K), jnp.bfloat16),
    )(codebooks)


def _fdk_body(q, m_ref, rsq_ref, cbt_ref, cbn_ref, idx_ref, ioff_ref,
              loss_ref):
    s = jnp.dot(m_ref[...].astype(jnp.bfloat16), cbt_ref[0],
                preferred_element_type=jnp.float32)
    dist = (rsq_ref[0] + 2.0 * s) + cbn_ref[0]
    mn = jnp.min(dist, axis=1, keepdims=True)
    ii = lax.broadcasted_iota(jnp.int32, (MTQ, K), 1)
    idx = jnp.min(jnp.where(dist == mn, ii, K), axis=1)
    idx_ref[0, 0, :] = idx
    ioff_ref[0, 0, :] = idx + q * K

    @pl.when(pl.program_id(0) == 0)
    def _():
        loss_ref[...] = jnp.zeros_like(loss_ref)

    loss_ref[...] += (jnp.sum(rsq_ref[0]) * (1.0 / (N_TOK * D_S))
                      ).reshape(1, 1)


def _fdku_body(q, m_ref, q_ref, rsq_ref, cbt_ref, cbn_ref, mn_ref,
               idx_ref, ioff_ref, loss_ref):
    mn = m_ref[...] + q_ref[...]
    mn_ref[...] = mn
    s = jnp.dot(mn.astype(jnp.bfloat16), cbt_ref[0],
                preferred_element_type=jnp.float32)
    dist = (rsq_ref[0] + 2.0 * s) + cbn_ref[0]
    mnv = jnp.min(dist, axis=1, keepdims=True)
    ii = lax.broadcasted_iota(jnp.int32, (MTQ, K), 1)
    idx = jnp.min(jnp.where(dist == mnv, ii, K), axis=1)
    idx_ref[0, 0, :] = idx
    ioff_ref[0, 0, :] = idx + q * K

    @pl.when(pl.program_id(0) == 0)
    def _():
        loss_ref[...] = jnp.zeros_like(loss_ref)

    loss_ref[...] += (jnp.sum(rsq_ref[0]) * (1.0 / (N_TOK * D_S))
                      ).reshape(1, 1)


def _fdk(m, rsq, cbt, cbn, q):
    return pl.pallas_call(
        lambda *a: _fdk_body(q, *a),
        grid=(NMTQ,),
        in_specs=[
            pl.BlockSpec((MTQ, D_S), lambda i: (i, 0)),
            pl.BlockSpec((1, MTQ, 1), lambda i: (i, 0, 0)),
            pl.BlockSpec((1, D_S, K), lambda i, q=q: (q, 0, 0)),
            pl.BlockSpec((1, 1, K), lambda i, q=q: (q, 0, 0)),
        ],
        out_specs=[
            pl.BlockSpec((1, 1, MTQ), lambda i: (i, 0, 0)),
            pl.BlockSpec((1, 1, MTQ), lambda i: (i, 0, 0)),
            pl.BlockSpec((1, 1), lambda i: (0, 0)),
        ],
        out_shape=[
            jax.ShapeDtypeStruct((NMTQ, 1, MTQ), jnp.int32),
            jax.ShapeDtypeStruct((NMTQ, 1, MTQ), jnp.int32),
            jax.ShapeDtypeStruct((1, 1), jnp.float32),
        ],
    )(m, rsq, cbt, cbn)


def _fdku(m_prev, quant, rsq, cbt, cbn, q):
    return pl.pallas_call(
        lambda *a: _fdku_body(q, *a),
        grid=(NMTQ,),
        in_specs=[
            pl.BlockSpec((MTQ, D_S), lambda i: (i, 0)),
            pl.BlockSpec((MTQ, D_S), lambda i: (i, 0)),
            pl.BlockSpec((1, MTQ, 1), lambda i: (i, 0, 0)),
            pl.BlockSpec((1, D_S, K), lambda i, q=q: (q, 0, 0)),
            pl.BlockSpec((1, 1, K), lambda i, q=q: (q, 0, 0)),
        ],
        out_specs=[
            pl.BlockSpec((MTQ, D_S), lambda i: (i, 0)),
            pl.BlockSpec((1, 1, MTQ), lambda i: (i, 0, 0)),
            pl.BlockSpec((1, 1, MTQ), lambda i: (i, 0, 0)),
            pl.BlockSpec((1, 1), lambda i: (0, 0)),
        ],
        out_shape=[
            jax.ShapeDtypeStruct((N_TOK, D_S), jnp.float32),
            jax.ShapeDtypeStruct((NMTQ, 1, MTQ), jnp.int32),
            jax.ShapeDtypeStruct((NMTQ, 1, MTQ), jnp.int32),
            jax.ShapeDtypeStruct((1, 1), jnp.float32),
        ],
    )(m_prev, quant, rsq, cbt, cbn)


def _scupd_body(cb_hbm, ioff_hbm, out_hbm, idx_v, rows_v, sem):
    wid = lax.axis_index("s") * SC_NC + lax.axis_index("c")
    base = wid * ROWS_PER_W
    pltpu.sync_copy(ioff_hbm.at[pl.ds(base, ROWS_PER_W)], idx_v)
    pltpu.async_copy(cb_hbm.at[idx_v], rows_v, sem).wait()
    pltpu.sync_copy(rows_v, out_hbm.at[pl.ds(base, ROWS_PER_W)])


_SC_UPD_KERNEL = None


def _sc_gather(cb_flat, ioff):
    global _SC_UPD_KERNEL
    if _SC_UPD_KERNEL is None:
        _SC_UPD_KERNEL = pl.kernel(
            _scupd_body,
            out_type=jax.ShapeDtypeStruct((N_TOK, D_S), jnp.float32),
            mesh=plsc.VectorSubcoreMesh(core_axis_name="c",
                                        subcore_axis_name="s",
                                        num_cores=SC_NC,
                                        num_subcores=SC_NS),
            scratch_types=[
                pltpu.VMEM((ROWS_PER_W,), jnp.int32),
                pltpu.VMEM((ROWS_PER_W, D_S), jnp.float32),
                pltpu.SemaphoreType.DMA,
            ],
        )
    return _SC_UPD_KERNEL(cb_flat, ioff)


def _decoder_body(z_ref, m_ref, q_ref, w3_ref, b3_ref, w4_ref, b4_ref,
                  out_ref, loss_ref):
    m = m_ref[...] + q_ref[...]

    @pl.when(pl.program_id(0) == 0)
    def _():
        loss_ref[...] = jnp.zeros_like(loss_ref)

    loss_ref[...] += (jnp.sum(m * m) * (1.0 / (N_TOK * D_S))).reshape(1, 1)
    zq = (z_ref[...] + m).astype(jnp.bfloat16)
    h = jnp.dot(zq, w3_ref[...], preferred_element_type=jnp.float32)
    h = h + b3_ref[...]
    o = jnp.dot(h.astype(jnp.bfloat16), w4_ref[...],
                preferred_element_type=jnp.float32)
    out_ref[...] = o + b4_ref[...]


def _decoder(z_e, m_last, quant_last, w3_16, b3, w4_16, b4):
    return pl.pallas_call(
        _decoder_body,
        grid=(NMT,),
        in_specs=[
            pl.BlockSpec((MT, D_S), lambda i: (i, 0)),
            pl.BlockSpec((MT, D_S), lambda i: (i, 0)),
            pl.BlockSpec((MT, D_S), lambda i: (i, 0)),
            pl.BlockSpec((D_S, D_F), lambda i: (0, 0)),
            pl.BlockSpec((1, D_F), lambda i: (0, 0)),
            pl.BlockSpec((D_F, D_EMB), lambda i: (0, 0)),
            pl.BlockSpec((1, D_EMB), lambda i: (0, 0)),
        ],
        out_specs=[
            pl.BlockSpec((MT, D_EMB), lambda i: (i, 0)),
            pl.BlockSpec((1, 1), lambda i: (0, 0)),
        ],
        out_shape=[
            jax.ShapeDtypeStruct((N_TOK, D_EMB), jnp.float32),
            jax.ShapeDtypeStruct((1, 1), jnp.float32),
        ],
    )(z_e, m_last, quant_last, w3_16, b3, w4_16, b4)


def _rsq_sum(m_prev, quant):
    return jnp.sum((m_prev + quant).reshape(1, N_TOK, D_S) ** 2, axis=-1,
                   keepdims=True).reshape(NMTQ, MTQ, 1)


def _rsq(m):
    return jnp.sum(m.reshape(1, N_TOK, D_S) ** 2, axis=-1,
                   keepdims=True).reshape(NMTQ, MTQ, 1)


def kernel(x, W1, b1, W2, b2, codebooks, W3, b3, W4, b4):
    z_e, m = _encoder(x.reshape(N_TOK, D_EMB),
                      W1.astype(jnp.bfloat16), b1.reshape(1, D_F),
                      W2.astype(jnp.bfloat16), b2.reshape(1, D_S))

    cbt = _prep(codebooks)
    cbn = jnp.sum(codebooks ** 2, axis=-1).reshape(NQ, 1, K)
    cb_flat = codebooks.reshape(NQ * K, D_S)

    idxs = []
    losses = []
    idx0, ioff0, _ = _fdk(m, _rsq(m), cbt, cbn, 0)
    idxs.append(idx0.reshape(N_TOK))
    quant = _sc_gather(cb_flat, ioff0.reshape(N_TOK))
    for q in range(1, NQ):
        rsq = _rsq_sum(m, quant)
        m, idx_q, ioff_q, loss_q = _fdku(m, quant, rsq, cbt, cbn, q)
        idxs.append(idx_q.reshape(N_TOK))
        losses.append(loss_q)
        quant = _sc_gather(cb_flat, ioff_q.reshape(N_TOK))

    out2, loss_last = _decoder(z_e, m, quant,
                               W3.astype(jnp.bfloat16), b3.reshape(1, D_F),
                               W4.astype(jnp.bfloat16), b4.reshape(1, D_EMB))
    losses.append(loss_last)

    out = out2.reshape(1, N_TOK, D_EMB)
    indices = jnp.stack([i.reshape(1, N_TOK) for i in idxs], axis=-1)
    cmt_loss = jnp.concatenate([l.reshape(1) for l in losses])
    return (out, indices, cmt_loss)
